# baseline (speedup 1.0000x reference)
import jax
import jax.numpy as jnp
from jax.experimental import pallas as pl

N = 10000
E = 320000
D = 128
H = 8
C = 16
B = 3
ALPHA = 0.2


def _graph_norm(x, w, b, ms, eps=1e-5):
    mean = jnp.mean(x, axis=0, keepdims=True)
    out = x - ms * mean
    var = jnp.mean(out * out, axis=0, keepdims=True)
    return w * out / jnp.sqrt(var + eps) + b


def _segment_softmax(logits, seg, num):
    m = jax.ops.segment_max(logits, seg, num_segments=num)
    m = jnp.where(jnp.isfinite(m), m, 0.0)
    ex = jnp.exp(logits - m[seg])
    s = jax.ops.segment_sum(ex, seg, num_segments=num)
    return ex / (s[seg] + 1e-16)


def _gatv2(x, src, dst, ew, eattr, Wl, Wr, We, att, b):
    xl = (x @ Wl).reshape(N, H, C)
    xr = (x @ Wr).reshape(N, H, C)
    e = xl[src] + xr[dst] + (eattr @ We).reshape(E, H, C)
    e = jax.nn.leaky_relu(e, 0.2)
    logits = jnp.sum(e * att, axis=-1)
    a = _segment_softmax(logits, dst, N)
    a = a * ew[:, None]
    msg = a[:, :, None] * xl[src]
    out = jax.ops.segment_sum(msg, dst, num_segments=N)
    return out.reshape(N, H * C) + b


def kernel(x, edge_index, edge_weight, edge_code, edge_table, pn_w, pn_b, pn_ms, Wl, Wr, We, att_p, gat_b, nw, nb, nms):
    src = edge_index[0]
    dst = edge_index[1]
    eattr = edge_table[edge_code]
    x0 = _graph_norm(x, pn_w, pn_b, pn_ms)
    h = x0
    for i in range(B):
        hn = _graph_norm(h, nw[i], nb[i], nms[i])
        hhat = _gatv2(hn, src, dst, edge_weight, eattr, Wl[i], Wr[i], We[i], att_p[i], gat_b[i])
        hhat = jax.nn.leaky_relu(hhat, 0.01)
        h = ALPHA * x0 + (1.0 - ALPHA) * hhat
    probe = pl.pallas_call(
        lambda a_ref, o_ref: o_ref.__setitem__((...,), a_ref[...] * 1.0),
        out_shape=jax.ShapeDtypeStruct((8, 128), jnp.float32),
    )(h[:8, :])
    return h.at[:8, :].set(probe)


# speedup vs baseline: 8.4805x; 8.4805x over previous
import functools

import jax
import jax.numpy as jnp
from jax import lax
from jax.experimental import pallas as pl
from jax.experimental.pallas import tpu as pltpu
from jax.experimental.pallas import tpu_sc as plsc

N = 10000
E = 320000
D = 128
H = 8
C = 16
B = 3
EV = 32
ED = 16
ALPHA = 0.2

NC = 2
NS = 16
NW = NC * NS
K = 128
EP = 327680
PW = EP // NW
NCHUNK = PW // K
EB = 1280
GB = EP // EB
NP = 10240
NTC = NP // K // NS
HN = NP // 2
CH = EP // K
CHT = CH // NS
HC128 = HN // K

_f32 = jnp.float32
_HIGH = lax.Precision.HIGHEST


def _dot(a, b):
    return jnp.dot(a, b, precision=_HIGH, preferred_element_type=_f32)


def _graph_norm(x, w, b, ms, eps=1e-5):
    mean = jnp.mean(x, axis=0, keepdims=True)
    out = x - ms * mean
    var = jnp.mean(out * out, axis=0, keepdims=True)
    return w * out / jnp.sqrt(var + eps) + b


def _tc_nm0_body(x_ref, pnw, pnb, pnms, nwv, nbv, nmsv, wl, wr, ettab, we,
                 x0_ref, xl_ref, xr_ref, et_ref):
    x0 = _graph_norm(x_ref[...], pnw[...], pnb[...], pnms[...])
    hn = _graph_norm(x0, nwv[...], nbv[...], nmsv[...])
    x0_ref[...] = x0
    xl_ref[...] = _dot(hn, wl[...])
    xr_ref[...] = _dot(hn, wr[...])
    et_ref[...] = _dot(ettab[...], we[...])


def _tc_fin_body(u2_ref, s2_ref, bias, x0_ref, h_ref):
    u = u2_ref[0:N]
    s = s2_ref[0:N, 0:H]
    r16 = lax.broadcasted_iota(jnp.int32, (H, D), 0)
    c16 = lax.broadcasted_iota(jnp.int32, (H, D), 1)
    sel = (r16 == c16 // C).astype(_f32)
    sfull = _dot(s, sel)
    hhat = u / (sfull + 1e-16) + bias[...]
    hhat = jnp.where(hhat >= 0, hhat, 0.01 * hhat)
    h_ref[...] = ALPHA * x0_ref[...] + (1.0 - ALPHA) * hhat


def _tc_nm_body(h_ref, nwv, nbv, nmsv, wl, wr, ettab, we,
                xl_ref, xr_ref, et_ref):
    hn = _graph_norm(h_ref[...], nwv[...], nbv[...], nmsv[...])
    xl_ref[...] = _dot(hn, wl[...])
    xr_ref[...] = _dot(hn, wr[...])
    et_ref[...] = _dot(ettab[...], we[...])


def _tc_logits_body(gxl_ref, gxr_ref, get_ref, attf, em_ref, lg_ref,
                    srow_ref):
    t = gxl_ref[...] + gxr_ref[...] + get_ref[...]
    t = jnp.where(t >= 0, t, 0.2 * t)
    tw = t * attf[...]
    rA = lax.broadcasted_iota(jnp.int32, (D, H), 0)
    cA = lax.broadcasted_iota(jnp.int32, (D, H), 1)
    selA = (rA // C == cA).astype(_f32)
    lg = _dot(tw, selA)
    lg_ref[...] = lg
    mask8 = em_ref[:, H:16]
    r1 = lax.broadcasted_iota(jnp.int32, (H, D), 0)
    c1 = lax.broadcasted_iota(jnp.int32, (H, D), 1)
    p1 = (c1 == r1).astype(_f32)
    p2 = (c1 == r1 + H).astype(_f32)
    p3 = (c1 == r1 + 2 * H).astype(_f32)
    lgm = lg * mask8
    srow_ref[...] = _dot(lgm, p1) + _dot(lgm * lg, p2) + _dot(mask8, p3)


def _tc_v_body(st2_ref, v_ref):
    st = st2_ref[...]
    dg = st[:, 2 * H:3 * H]
    mu = st[:, 0:H] / jnp.maximum(dg, 1.0)
    msq = st[:, H:2 * H] / jnp.maximum(dg, 1.0)
    var = msq - mu * mu
    sig = jnp.sqrt(jnp.maximum(var, 0.0))
    v8 = jnp.where(dg > 0, mu + 2.5 * sig + 2.0, 0.0)
    r1 = lax.broadcasted_iota(jnp.int32, (H, D), 0)
    c1 = lax.broadcasted_iota(jnp.int32, (H, D), 1)
    p1 = (c1 == r1).astype(_f32)
    v_ref[...] = _dot(v8, p1)


def _tc_exp_body(lg_ref, gv_ref, em_ref, gxl_ref, msg_ref, srow2_ref):
    ex = jnp.exp(lg_ref[...] - gv_ref[:, 0:H])
    em = em_ref[...]
    r1 = lax.broadcasted_iota(jnp.int32, (H, D), 0)
    c1 = lax.broadcasted_iota(jnp.int32, (H, D), 1)
    p1 = (c1 == r1).astype(_f32)
    srow2_ref[...] = _dot(ex * em[:, H:16], p1)
    p = ex * em[:, 0:H]
    selT = (c1 // C == r1).astype(_f32)
    msg_ref[...] = _dot(p, selT) * gxl_ref[...]


_MESH = plsc.VectorSubcoreMesh(core_axis_name="c", subcore_axis_name="s")


def _wid_base():
    cid = lax.axis_index("c")
    sid = lax.axis_index("s")
    return cid, sid, (sid * NC + cid) * PW


@functools.partial(
    pl.kernel,
    out_type=(
        jax.ShapeDtypeStruct((EP, D), _f32),
        jax.ShapeDtypeStruct((EP, D), _f32),
        jax.ShapeDtypeStruct((EP, D), _f32),
    ),
    mesh=_MESH,
    scratch_types=[
        pltpu.VMEM((K,), jnp.int32), pltpu.VMEM((K,), jnp.int32),
        pltpu.VMEM((K,), jnp.int32),
        pltpu.VMEM((K, D), _f32), pltpu.VMEM((K, D), _f32),
        pltpu.VMEM((K, D), _f32),
        pltpu.SemaphoreType.DMA, pltpu.SemaphoreType.DMA,
        pltpu.SemaphoreType.DMA,
    ],
)
def _sc_gather3(xl_hbm, xr_hbm, et_hbm, src_hbm, dst_hbm, code_hbm,
                gxl_hbm, gxr_hbm, get_hbm,
                si, di, ci, bl, br, bt, sem1, sem2, sem3):
    _, _, base = _wid_base()

    @pl.loop(0, NCHUNK)
    def _(ck):
        off = base + ck * K
        pltpu.sync_copy(src_hbm.at[pl.ds(off, K)], si)
        pltpu.sync_copy(dst_hbm.at[pl.ds(off, K)], di)
        pltpu.sync_copy(code_hbm.at[pl.ds(off, K)], ci)
        c1 = pltpu.async_copy(xl_hbm.at[si], bl, sem1)
        c2 = pltpu.async_copy(xr_hbm.at[di], br, sem2)
        c3 = pltpu.async_copy(et_hbm.at[ci], bt, sem3)
        c1.wait()
        c2.wait()
        c3.wait()
        pltpu.sync_copy(bl, gxl_hbm.at[pl.ds(off, K)])
        pltpu.sync_copy(br, gxr_hbm.at[pl.ds(off, K)])
        pltpu.sync_copy(bt, get_hbm.at[pl.ds(off, K)])


@functools.partial(
    pl.kernel,
    out_type=jax.ShapeDtypeStruct((NP, D), _f32),
    mesh=_MESH,
    scratch_types=[
        pltpu.VMEM((K,), jnp.int32),
        pltpu.VMEM((K, D), _f32),
        pltpu.VMEM((K, D), _f32),
        pltpu.VMEM_SHARED((HN + K, D), _f32),
    ],
)
def _sc_scatter128(rows_hbm, dst_hbm, zeros128_hbm, acc_out, di, mb, tb,
                   acc_sh):
    cid = lax.axis_index("c")
    sid = lax.axis_index("s")
    lo = cid * HN
    pltpu.sync_copy(zeros128_hbm, tb)

    @pl.loop(0, 3)
    def _(j):
        jj = j * NS + sid

        @pl.when(jj < HC128)
        def _():
            pltpu.sync_copy(tb, acc_sh.at[pl.ds(jj * K, K)])

    plsc.subcore_barrier()

    @pl.loop(0, CHT)
    def _(ck):
        off = (sid * CHT + ck) * K
        pltpu.sync_copy(dst_hbm.at[pl.ds(off, K)], di)
        pltpu.sync_copy(rows_hbm.at[pl.ds(off, K)], mb)

        @pl.loop(0, K // 16)
        def _(j):
            v = di[pl.ds(j * 16, 16)] - lo
            ok = (v >= 0) & (v < HN)
            di[pl.ds(j * 16, 16)] = jnp.where(ok, v, HN)

        pltpu.sync_copy(mb, acc_sh.at[di], add=True)

    plsc.subcore_barrier()

    @pl.loop(0, 3)
    def _(j):
        jj = j * NS + sid

        @pl.when(jj < HC128)
        def _():
            pltpu.sync_copy(acc_sh.at[pl.ds(jj * K, K)], tb)
            pltpu.sync_copy(tb, acc_out.at[pl.ds(lo + jj * K, K)])


@functools.partial(
    pl.kernel,
    out_type=jax.ShapeDtypeStruct((EP, D), _f32),
    mesh=_MESH,
    scratch_types=[
        pltpu.VMEM((K,), jnp.int32),
        pltpu.VMEM((K, D), _f32),
        pltpu.SemaphoreType.DMA,
    ],
)
def _sc_gather_v(v_hbm, dst_hbm, gv_hbm, di, vb, sem):
    _, _, base = _wid_base()

    @pl.loop(0, NCHUNK)
    def _(ck):
        off = base + ck * K
        pltpu.sync_copy(dst_hbm.at[pl.ds(off, K)], di)
        pltpu.async_copy(v_hbm.at[di], vb, sem).wait()
        pltpu.sync_copy(vb, gv_hbm.at[pl.ds(off, K)])


def _tc_call(body, out_shapes, *args):
    return pl.pallas_call(body, out_shape=out_shapes)(*args)


def kernel(x, edge_index, edge_weight, edge_code, edge_table, pn_w, pn_b,
           pn_ms, Wl, Wr, We, att_p, gat_b, nw, nb, nms):
    pad = EP - E
    src = jnp.concatenate([edge_index[0], jnp.zeros((pad,), jnp.int32)])
    dst = jnp.concatenate([edge_index[1], jnp.zeros((pad,), jnp.int32)])
    code = jnp.concatenate([edge_code, jnp.zeros((pad,), jnp.int32)])
    ewp = jnp.concatenate([edge_weight, jnp.zeros((pad,), _f32)])
    maskp = jnp.concatenate([jnp.ones((E,), _f32), jnp.zeros((pad,), _f32)])
    em = jnp.concatenate(
        [jnp.tile(ewp[:, None], (1, H)), jnp.tile(maskp[:, None], (1, H))],
        axis=1)

    zeros128 = jnp.zeros((K, D), _f32)

    row1 = lambda v: v.reshape(1, D)
    sds = jax.ShapeDtypeStruct

    x0, xl, xr, et = _tc_call(
        _tc_nm0_body,
        (sds((N, D), _f32), sds((N, D), _f32), sds((N, D), _f32),
         sds((EV, D), _f32)),
        x, row1(pn_w), row1(pn_b), row1(pn_ms),
        row1(nw[0]), row1(nb[0]), row1(nms[0]),
        Wl[0], Wr[0], edge_table, We[0])

    h = x0
    for i in range(B):
        if i > 0:
            xl, xr, et = _tc_call(
                _tc_nm_body,
                (sds((N, D), _f32), sds((N, D), _f32), sds((EV, D), _f32)),
                h, row1(nw[i]), row1(nb[i]), row1(nms[i]),
                Wl[i], Wr[i], edge_table, We[i])

        gxl, gxr, get = _sc_gather3(xl, xr, et, src, dst, code)

        attf = att_p[i].reshape(1, D)
        lg, srow = pl.pallas_call(
            _tc_logits_body,
            grid=(GB,),
            in_specs=[
                pl.BlockSpec((EB, D), lambda g: (g, 0)),
                pl.BlockSpec((EB, D), lambda g: (g, 0)),
                pl.BlockSpec((EB, D), lambda g: (g, 0)),
                pl.BlockSpec((1, D), lambda g: (0, 0)),
                pl.BlockSpec((EB, 16), lambda g: (g, 0)),
            ],
            out_specs=[
                pl.BlockSpec((EB, H), lambda g: (g, 0)),
                pl.BlockSpec((EB, D), lambda g: (g, 0)),
            ],
            out_shape=[sds((EP, H), _f32), sds((EP, D), _f32)],
        )(gxl, gxr, get, attf, em)

        st2 = _sc_scatter128(srow, dst, zeros128)

        vstat = _tc_call(_tc_v_body, sds((NP, D), _f32), st2)

        gv = _sc_gather_v(vstat, dst)

        msg, srow2 = pl.pallas_call(
            _tc_exp_body,
            grid=(GB,),
            in_specs=[
                pl.BlockSpec((EB, H), lambda g: (g, 0)),
                pl.BlockSpec((EB, D), lambda g: (g, 0)),
                pl.BlockSpec((EB, 16), lambda g: (g, 0)),
                pl.BlockSpec((EB, D), lambda g: (g, 0)),
            ],
            out_specs=[
                pl.BlockSpec((EB, D), lambda g: (g, 0)),
                pl.BlockSpec((EB, D), lambda g: (g, 0)),
            ],
            out_shape=[sds((EP, D), _f32), sds((EP, D), _f32)],
        )(lg, gv, em, gxl)

        u2 = _sc_scatter128(msg, dst, zeros128)
        s2 = _sc_scatter128(srow2, dst, zeros128)

        h = _tc_call(_tc_fin_body, sds((N, D), _f32),
                     u2, s2, gat_b[i].reshape(1, D), x0)

    return h


# speedup vs baseline: 9.2365x; 1.0891x over previous
import functools

import jax
import jax.numpy as jnp
from jax import lax
from jax.experimental import pallas as pl
from jax.experimental.pallas import tpu as pltpu
from jax.experimental.pallas import tpu_sc as plsc

N = 10000
E = 320000
D = 128
H = 8
C = 16
B = 3
EV = 32
ED = 16
ALPHA = 0.2

NC = 2
NS = 16
NW = NC * NS
K = 128
EP = 327680
PW = EP // NW
NCHUNK = PW // K
EB = 1280
GB = EP // EB
NP = 10240
NTC = NP // K // NS
HN = NP // 2
CH = EP // K
CHT = CH // NS
HC128 = HN // K

_f32 = jnp.float32
_HIGH = lax.Precision.HIGHEST


def _dot(a, b):
    return jnp.dot(a, b, precision=_HIGH, preferred_element_type=_f32)


def _graph_norm(x, w, b, ms, eps=1e-5):
    mean = jnp.mean(x, axis=0, keepdims=True)
    out = x - ms * mean
    var = jnp.mean(out * out, axis=0, keepdims=True)
    return w * out / jnp.sqrt(var + eps) + b


def _tc_nm0_body(x_ref, pnw, pnb, pnms, nwv, nbv, nmsv, wl, wr, ettab, we,
                 x0_ref, xl_ref, xr_ref, et_ref):
    x0 = _graph_norm(x_ref[...], pnw[...], pnb[...], pnms[...])
    hn = _graph_norm(x0, nwv[...], nbv[...], nmsv[...])
    x0_ref[...] = x0
    xl_ref[...] = _dot(hn, wl[...])
    xr_ref[...] = _dot(hn, wr[...])
    et_ref[...] = _dot(ettab[...], we[...])


def _tc_fin_body(u2_ref, s2_ref, bias, x0_ref, h_ref):
    u = u2_ref[0:N]
    s = s2_ref[0:N, 0:H]
    r16 = lax.broadcasted_iota(jnp.int32, (H, D), 0)
    c16 = lax.broadcasted_iota(jnp.int32, (H, D), 1)
    sel = (r16 == c16 // C).astype(_f32)
    sfull = _dot(s, sel)
    hhat = u / (sfull + 1e-16) + bias[...]
    hhat = jnp.where(hhat >= 0, hhat, 0.01 * hhat)
    h_ref[...] = ALPHA * x0_ref[...] + (1.0 - ALPHA) * hhat


def _tc_nm_body(h_ref, nwv, nbv, nmsv, wl, wr, ettab, we,
                xl_ref, xr_ref, et_ref):
    hn = _graph_norm(h_ref[...], nwv[...], nbv[...], nmsv[...])
    xl_ref[...] = _dot(hn, wl[...])
    xr_ref[...] = _dot(hn, wr[...])
    et_ref[...] = _dot(ettab[...], we[...])


def _tc_logits_body(gxl_ref, gxr_ref, get_ref, attf, em_ref, lg_ref,
                    srow_ref):
    t = gxl_ref[...] + gxr_ref[...] + get_ref[...]
    t = jnp.where(t >= 0, t, 0.2 * t)
    tw = t * attf[...]
    rA = lax.broadcasted_iota(jnp.int32, (D, H), 0)
    cA = lax.broadcasted_iota(jnp.int32, (D, H), 1)
    selA = (rA // C == cA).astype(_f32)
    lg = _dot(tw, selA)
    lg_ref[...] = lg
    mask8 = em_ref[:, H:16]
    r1 = lax.broadcasted_iota(jnp.int32, (H, D), 0)
    c1 = lax.broadcasted_iota(jnp.int32, (H, D), 1)
    p1 = (c1 == r1).astype(_f32)
    p2 = (c1 == r1 + H).astype(_f32)
    p3 = (c1 == r1 + 2 * H).astype(_f32)
    lgm = lg * mask8
    srow_ref[...] = _dot(lgm, p1) + _dot(lgm * lg, p2) + _dot(mask8, p3)


def _tc_v_body(st2_ref, v_ref):
    st = st2_ref[...]
    dg = st[:, 2 * H:3 * H]
    mu = st[:, 0:H] / jnp.maximum(dg, 1.0)
    msq = st[:, H:2 * H] / jnp.maximum(dg, 1.0)
    var = msq - mu * mu
    sig = jnp.sqrt(jnp.maximum(var, 0.0))
    v8 = jnp.where(dg > 0, mu + 2.5 * sig + 2.0, 0.0)
    r1 = lax.broadcasted_iota(jnp.int32, (H, D), 0)
    c1 = lax.broadcasted_iota(jnp.int32, (H, D), 1)
    p1 = (c1 == r1).astype(_f32)
    v_ref[...] = _dot(v8, p1)


def _tc_exp_body(lg_ref, gv_ref, em_ref, gxl_ref, msg_ref, srow2_ref):
    ex = jnp.exp(lg_ref[...] - gv_ref[:, 0:H])
    em = em_ref[...]
    r1 = lax.broadcasted_iota(jnp.int32, (H, D), 0)
    c1 = lax.broadcasted_iota(jnp.int32, (H, D), 1)
    p1 = (c1 == r1).astype(_f32)
    srow2_ref[...] = _dot(ex * em[:, H:16], p1)
    p = ex * em[:, 0:H]
    selT = (c1 // C == r1).astype(_f32)
    msg_ref[...] = _dot(p, selT) * gxl_ref[...]


_MESH = plsc.VectorSubcoreMesh(core_axis_name="c", subcore_axis_name="s")


def _wid_base():
    cid = lax.axis_index("c")
    sid = lax.axis_index("s")
    return cid, sid, (sid * NC + cid) * PW


@functools.partial(
    pl.kernel,
    out_type=(
        jax.ShapeDtypeStruct((EP, D), _f32),
        jax.ShapeDtypeStruct((EP, D), _f32),
        jax.ShapeDtypeStruct((EP, D), _f32),
    ),
    mesh=_MESH,
    scratch_types=[
        pltpu.VMEM((K,), jnp.int32), pltpu.VMEM((K,), jnp.int32),
        pltpu.VMEM((K,), jnp.int32),
        pltpu.VMEM((K, D), _f32), pltpu.VMEM((K, D), _f32),
        pltpu.VMEM((K, D), _f32),
        pltpu.SemaphoreType.DMA, pltpu.SemaphoreType.DMA,
        pltpu.SemaphoreType.DMA,
    ],
)
def _sc_gather3(xl_hbm, xr_hbm, et_hbm, src_hbm, dst_hbm, code_hbm,
                gxl_hbm, gxr_hbm, get_hbm,
                si, di, ci, bl, br, bt, sem1, sem2, sem3):
    _, _, base = _wid_base()

    @pl.loop(0, NCHUNK)
    def _(ck):
        off = base + ck * K
        pltpu.sync_copy(src_hbm.at[pl.ds(off, K)], si)
        pltpu.sync_copy(dst_hbm.at[pl.ds(off, K)], di)
        pltpu.sync_copy(code_hbm.at[pl.ds(off, K)], ci)
        c1 = pltpu.async_copy(xl_hbm.at[si], bl, sem1)
        c2 = pltpu.async_copy(xr_hbm.at[di], br, sem2)
        c3 = pltpu.async_copy(et_hbm.at[ci], bt, sem3)
        c1.wait()
        c2.wait()
        c3.wait()
        pltpu.sync_copy(bl, gxl_hbm.at[pl.ds(off, K)])
        pltpu.sync_copy(br, gxr_hbm.at[pl.ds(off, K)])
        pltpu.sync_copy(bt, get_hbm.at[pl.ds(off, K)])


@functools.partial(
    pl.kernel,
    out_type=jax.ShapeDtypeStruct((NP, D), _f32),
    mesh=_MESH,
    scratch_types=[
        pltpu.VMEM((K,), jnp.int32), pltpu.VMEM((K,), jnp.int32),
        pltpu.VMEM((K, D), _f32), pltpu.VMEM((K, D), _f32),
        pltpu.VMEM((K, D), _f32),
        pltpu.VMEM_SHARED((HN + K, D), _f32),
        pltpu.SemaphoreType.DMA, pltpu.SemaphoreType.DMA,
    ],
)
def _sc_scatter128(rows_hbm, dst_hbm, zeros128_hbm, acc_out, di0, di1,
                   mb0, mb1, tb, acc_sh, sl0, sl1):
    cid = lax.axis_index("c")
    sid = lax.axis_index("s")
    lo = cid * HN
    pltpu.sync_copy(zeros128_hbm, tb)

    @pl.loop(0, 3)
    def _(j):
        jj = j * NS + sid

        @pl.when(jj < HC128)
        def _():
            pltpu.sync_copy(tb, acc_sh.at[pl.ds(jj * K, K)])

    plsc.subcore_barrier()
    base = sid * CHT

    def _issue(cc, di, mb, sem):
        pltpu.async_copy(dst_hbm.at[pl.ds(cc * K, K)], di, sem)
        pltpu.async_copy(rows_hbm.at[pl.ds(cc * K, K)], mb, sem)

    def _drain(di, mb, sem):
        pltpu.make_async_copy(dst_hbm.at[pl.ds(0, K)], di, sem).wait()
        pltpu.make_async_copy(rows_hbm.at[pl.ds(0, K)], mb, sem).wait()

    def _remap_scatter(di, mb):
        @pl.loop(0, K // 16)
        def _(j):
            v = di[pl.ds(j * 16, 16)] - lo
            ok = (v >= 0) & (v < HN)
            di[pl.ds(j * 16, 16)] = jnp.where(ok, v, HN)

        pltpu.sync_copy(mb, acc_sh.at[di], add=True)

    _issue(base, di0, mb0, sl0)

    @pl.loop(0, CHT // 2)
    def _(j2):
        cc = base + 2 * j2
        _issue(cc + 1, di1, mb1, sl1)
        _drain(di0, mb0, sl0)
        _remap_scatter(di0, mb0)

        @pl.when(j2 < CHT // 2 - 1)
        def _():
            _issue(cc + 2, di0, mb0, sl0)

        _drain(di1, mb1, sl1)
        _remap_scatter(di1, mb1)

    plsc.subcore_barrier()

    @pl.loop(0, 3)
    def _(j):
        jj = j * NS + sid

        @pl.when(jj < HC128)
        def _():
            pltpu.sync_copy(acc_sh.at[pl.ds(jj * K, K)], tb)
            pltpu.sync_copy(tb, acc_out.at[pl.ds(lo + jj * K, K)])


@functools.partial(
    pl.kernel,
    out_type=jax.ShapeDtypeStruct((EP, D), _f32),
    mesh=_MESH,
    scratch_types=[
        pltpu.VMEM((K,), jnp.int32),
        pltpu.VMEM((K, D), _f32),
        pltpu.SemaphoreType.DMA,
    ],
)
def _sc_gather_v(v_hbm, dst_hbm, gv_hbm, di, vb, sem):
    _, _, base = _wid_base()

    @pl.loop(0, NCHUNK)
    def _(ck):
        off = base + ck * K
        pltpu.sync_copy(dst_hbm.at[pl.ds(off, K)], di)
        pltpu.async_copy(v_hbm.at[di], vb, sem).wait()
        pltpu.sync_copy(vb, gv_hbm.at[pl.ds(off, K)])


def _tc_call(body, out_shapes, *args):
    return pl.pallas_call(body, out_shape=out_shapes)(*args)


def kernel(x, edge_index, edge_weight, edge_code, edge_table, pn_w, pn_b,
           pn_ms, Wl, Wr, We, att_p, gat_b, nw, nb, nms):
    pad = EP - E
    src = jnp.concatenate([edge_index[0], jnp.zeros((pad,), jnp.int32)])
    dst = jnp.concatenate([edge_index[1], jnp.zeros((pad,), jnp.int32)])
    code = jnp.concatenate([edge_code, jnp.zeros((pad,), jnp.int32)])
    ewp = jnp.concatenate([edge_weight, jnp.zeros((pad,), _f32)])
    maskp = jnp.concatenate([jnp.ones((E,), _f32), jnp.zeros((pad,), _f32)])
    em = jnp.concatenate(
        [jnp.tile(ewp[:, None], (1, H)), jnp.tile(maskp[:, None], (1, H))],
        axis=1)

    zeros128 = jnp.zeros((K, D), _f32)

    row1 = lambda v: v.reshape(1, D)
    sds = jax.ShapeDtypeStruct

    x0, xl, xr, et = _tc_call(
        _tc_nm0_body,
        (sds((N, D), _f32), sds((N, D), _f32), sds((N, D), _f32),
         sds((EV, D), _f32)),
        x, row1(pn_w), row1(pn_b), row1(pn_ms),
        row1(nw[0]), row1(nb[0]), row1(nms[0]),
        Wl[0], Wr[0], edge_table, We[0])

    h = x0
    for i in range(B):
        if i > 0:
            xl, xr, et = _tc_call(
                _tc_nm_body,
                (sds((N, D), _f32), sds((N, D), _f32), sds((EV, D), _f32)),
                h, row1(nw[i]), row1(nb[i]), row1(nms[i]),
                Wl[i], Wr[i], edge_table, We[i])

        gxl, gxr, get = _sc_gather3(xl, xr, et, src, dst, code)

        attf = att_p[i].reshape(1, D)
        lg, srow = pl.pallas_call(
            _tc_logits_body,
            grid=(GB,),
            in_specs=[
                pl.BlockSpec((EB, D), lambda g: (g, 0)),
                pl.BlockSpec((EB, D), lambda g: (g, 0)),
                pl.BlockSpec((EB, D), lambda g: (g, 0)),
                pl.BlockSpec((1, D), lambda g: (0, 0)),
                pl.BlockSpec((EB, 16), lambda g: (g, 0)),
            ],
            out_specs=[
                pl.BlockSpec((EB, H), lambda g: (g, 0)),
                pl.BlockSpec((EB, D), lambda g: (g, 0)),
            ],
            out_shape=[sds((EP, H), _f32), sds((EP, D), _f32)],
        )(gxl, gxr, get, attf, em)

        st2 = _sc_scatter128(srow, dst, zeros128)

        vstat = _tc_call(_tc_v_body, sds((NP, D), _f32), st2)

        gv = _sc_gather_v(vstat, dst)

        msg, srow2 = pl.pallas_call(
            _tc_exp_body,
            grid=(GB,),
            in_specs=[
                pl.BlockSpec((EB, H), lambda g: (g, 0)),
                pl.BlockSpec((EB, D), lambda g: (g, 0)),
                pl.BlockSpec((EB, 16), lambda g: (g, 0)),
                pl.BlockSpec((EB, D), lambda g: (g, 0)),
            ],
            out_specs=[
                pl.BlockSpec((EB, D), lambda g: (g, 0)),
                pl.BlockSpec((EB, D), lambda g: (g, 0)),
            ],
            out_shape=[sds((EP, D), _f32), sds((EP, D), _f32)],
        )(lg, gv, em, gxl)

        u2 = _sc_scatter128(msg, dst, zeros128)
        s2 = _sc_scatter128(srow2, dst, zeros128)

        h = _tc_call(_tc_fin_body, sds((N, D), _f32),
                     u2, s2, gat_b[i].reshape(1, D), x0)

    return h
